# Initial kernel scaffold; baseline (speedup 1.0000x reference)
import jax
import jax.numpy as jnp
from jax.experimental import pallas as pl


def kernel(x, opcode_onehot, W_up, b_up, W_gate, b_gate, W_down, b_down):
    raise NotImplementedError("write your pallas kernel here")


# speedup vs baseline: 3.6385x; 3.6385x over previous
import jax
import jax.numpy as jnp
from jax.experimental import pallas as pl
from jax.experimental.pallas import tpu as pltpu

_B, _S, _D, _H, _E = 2, 2048, 1024, 4096, 8
_BS = 1024
_BH = 1024


def _route_kernel(oh_ref, active_ref):
    for i in range(_B):
        best = oh_ref[i, 0]
        besti = jnp.int32(0)
        for e in range(1, _E):
            v = oh_ref[i, e]
            pred = v > best
            best = jnp.where(pred, v, best)
            besti = jnp.where(pred, jnp.int32(e), besti)
        active_ref[i] = besti


def _ffn_kernel(act_ref, x_ref, wu_ref, wg_ref, wd_ref, bu_ref, bg_ref,
                bd_ref, o_ref):
    h = pl.program_id(2)
    x = x_ref[0].astype(jnp.bfloat16)
    nt = (((1,), (1,)), ((), ()))
    up = jax.lax.dot_general(x, wu_ref[0].astype(jnp.bfloat16), nt,
                             preferred_element_type=jnp.float32)
    up = up + bu_ref[0]
    gate = jax.lax.dot_general(x, wg_ref[0].astype(jnp.bfloat16), nt,
                               preferred_element_type=jnp.float32)
    gate = gate + bg_ref[0]
    hidden = (up * jax.lax.logistic(up) * gate).astype(jnp.bfloat16)
    contrib = jax.lax.dot_general(hidden, wd_ref[0].astype(jnp.bfloat16), nt,
                                  preferred_element_type=jnp.float32)

    @pl.when(h == 0)
    def _():
        o_ref[0] = contrib + bd_ref[0]

    @pl.when(h != 0)
    def _():
        o_ref[0] += contrib


def kernel(x, opcode_onehot, W_up, b_up, W_gate, b_gate, W_down, b_down):
    active = pl.pallas_call(
        _route_kernel,
        in_specs=[pl.BlockSpec(memory_space=pltpu.SMEM)],
        out_specs=pl.BlockSpec(memory_space=pltpu.SMEM),
        out_shape=jax.ShapeDtypeStruct((_B,), jnp.int32),
    )(opcode_onehot)

    bu = b_up.reshape(_E, 1, _H)
    bg = b_gate.reshape(_E, 1, _H)
    bd = b_down.reshape(_E, 1, _D)

    grid = (_B, _S // _BS, _H // _BH)
    grid_spec = pltpu.PrefetchScalarGridSpec(
        num_scalar_prefetch=1,
        grid=grid,
        in_specs=[
            pl.BlockSpec((1, _BS, _D), lambda b, s, h, act: (b, s, 0)),
            pl.BlockSpec((1, _BH, _D), lambda b, s, h, act: (act[b], h, 0)),
            pl.BlockSpec((1, _BH, _D), lambda b, s, h, act: (act[b], h, 0)),
            pl.BlockSpec((1, _D, _BH), lambda b, s, h, act: (act[b], 0, h)),
            pl.BlockSpec((1, 1, _BH), lambda b, s, h, act: (act[b], 0, h)),
            pl.BlockSpec((1, 1, _BH), lambda b, s, h, act: (act[b], 0, h)),
            pl.BlockSpec((1, 1, _D), lambda b, s, h, act: (act[b], 0, 0)),
        ],
        out_specs=pl.BlockSpec((1, _BS, _D), lambda b, s, h, act: (b, s, 0)),
    )
    out = pl.pallas_call(
        _ffn_kernel,
        grid_spec=grid_spec,
        out_shape=jax.ShapeDtypeStruct((_B, _S, _D), jnp.float32),
        compiler_params=pltpu.CompilerParams(
            dimension_semantics=("parallel", "parallel", "arbitrary"),
        ),
    )(active, x, W_up, W_gate, W_down, bu, bg, bd)
    return out
